# baseline (speedup 1.0000x reference)
import functools

import jax
import jax.numpy as jnp
from jax import lax
from jax.experimental import pallas as pl
from jax.experimental.pallas import tpu as pltpu
from jax.experimental.pallas import tpu_sc as plsc

NUM_DEEP = 26
VOCAB = 100000
EMB = 64
B = 16384
WAD = 128
H = WAD // 2
HIDDEN = H * 2
EPS = 1e-6

NTOT = NUM_DEEP * B
NW = 32
ROWS_PER_W = NTOT // NW
CHUNK = 128
NCHUNK = ROWS_PER_W // CHUNK

_mesh = plsc.VectorSubcoreMesh(core_axis_name="c", subcore_axis_name="s")


@functools.partial(
    pl.kernel,
    mesh=_mesh,
    out_type=jax.ShapeDtypeStruct((NTOT, EMB), jnp.float32),
    scratch_types=[
        pltpu.VMEM((NCHUNK, CHUNK), jnp.int32),
        pltpu.VMEM((CHUNK, EMB), jnp.float32),
        pltpu.SemaphoreType.DMA,
    ],
    compiler_params=pltpu.CompilerParams(use_tc_tiling_on_sc=False),
)
def _sc_gather(tables_hbm, idx_hbm, out_hbm, idx_v, rows_v, sem):
    wid = lax.axis_index("s") * 2 + lax.axis_index("c")
    base = wid * ROWS_PER_W
    pltpu.sync_copy(idx_hbm.at[pl.ds(wid * NCHUNK, NCHUNK)], idx_v)

    def body(j, carry):
        pltpu.async_copy(tables_hbm.at[idx_v.at[j]], rows_v, sem).wait()
        pltpu.sync_copy(rows_v, out_hbm.at[pl.ds(base + j * CHUNK, CHUNK)])
        return carry

    lax.fori_loop(0, NCHUNK, body, 0)


def _tc_body(x_ref, wd_ref, bd_ref, w13_ref, b13_ref, w2_ref, b2_ref, o_ref):
    x = x_ref[...]
    ss = jnp.sum(x * x, axis=1, keepdims=True) * (1.0 / (NUM_DEEP * EMB))
    t = jnp.dot(x, wd_ref[...], preferred_element_type=jnp.float32)
    h = t * lax.rsqrt(ss + EPS) + bd_ref[...]
    gu = jnp.dot(h, w13_ref[...], preferred_element_type=jnp.float32) + b13_ref[...]
    g = gu[:, :HIDDEN]
    u = gu[:, HIDDEN:]
    act = g * jax.nn.sigmoid(g) * u
    o_ref[...] = jnp.dot(act, w2_ref[...], preferred_element_type=jnp.float32) + b2_ref[...]


BBLK = 2048


def _tc_head(x, wd_s, bd, w13, b13, w2, b2):
    grid = (B // BBLK,)
    full = lambda shape: pl.BlockSpec(shape, lambda i: (0, 0))
    return pl.pallas_call(
        _tc_body,
        grid=grid,
        in_specs=[
            pl.BlockSpec((BBLK, NUM_DEEP * EMB), lambda i: (i, 0)),
            full((NUM_DEEP * EMB, H)),
            full((1, H)),
            full((H, 2 * HIDDEN)),
            full((1, 2 * HIDDEN)),
            full((HIDDEN, H)),
            full((1, H)),
        ],
        out_specs=pl.BlockSpec((BBLK, H), lambda i: (i, 0)),
        out_shape=jax.ShapeDtypeStruct((B, H), jnp.float32),
    )(x, wd_s, bd, w13, b13, w2, b2)


def kernel(deep_in, tables, rms_w, Wd, bd, W1, b1, W3, b3, W2, b2):
    tables_flat = tables.reshape(NUM_DEEP * VOCAB, EMB)
    offs = (jnp.arange(NUM_DEEP, dtype=jnp.int32) * VOCAB)[:, None]
    fidx = (deep_in + offs).T.reshape(NTOT // CHUNK, CHUNK)
    gathered = _sc_gather(tables_flat, fidx)
    x = gathered.reshape(B, NUM_DEEP * EMB)
    wd_s = rms_w[:, None] * Wd
    w13 = jnp.concatenate([W1, W3], axis=1)
    b13 = jnp.concatenate([b1, b3]).reshape(1, -1)
    return _tc_head(x, wd_s, bd.reshape(1, H), w13, b13, W2, b2.reshape(1, H))
